# baseline (speedup 1.0000x reference)
import functools

import jax
import jax.numpy as jnp
from jax import lax
from jax.experimental import pallas as pl
from jax.experimental.pallas import tpu as pltpu
from jax.experimental.pallas import tpu_sc as plsc


def _mm_stats_body(x1_ref, x2_ref, wup_ref, wlat_ref,
                   h_ref, lat_ref, s1_ref, ss1_ref, s2_ref, ss2_ref):
    b = pl.program_id(0)
    h = jnp.dot(x1_ref[0], wup_ref[...], preferred_element_type=jnp.float32)
    lat = jnp.dot(x2_ref[0], wlat_ref[...], preferred_element_type=jnp.float32)
    h_ref[0] = h
    lat_ref[0] = lat
    s1 = jnp.sum(h, axis=0, keepdims=True)
    ss1 = jnp.sum(h * h, axis=0, keepdims=True)
    s2 = jnp.sum(lat, axis=0, keepdims=True)
    ss2 = jnp.sum(lat * lat, axis=0, keepdims=True)

    @pl.when(b == 0)
    def _():
        s1_ref[...] = s1
        ss1_ref[...] = ss1
        s2_ref[...] = s2
        ss2_ref[...] = ss2

    @pl.when(b != 0)
    def _():
        s1_ref[...] += s1
        ss1_ref[...] += ss1
        s2_ref[...] += s2
        ss2_ref[...] += ss2


def _mm_stats(x1, x2, w_up_t, w_lat_t):
    B, N, Cin = x1.shape
    M = x2.shape[1]
    Cout = w_up_t.shape[1]
    return pl.pallas_call(
        _mm_stats_body,
        grid=(B,),
        in_specs=[
            pl.BlockSpec((1, N, Cin), lambda b: (b, 0, 0)),
            pl.BlockSpec((1, M, Cout), lambda b: (b, 0, 0)),
            pl.BlockSpec((Cin, Cout), lambda b: (0, 0)),
            pl.BlockSpec((Cout, Cout), lambda b: (0, 0)),
        ],
        out_specs=[
            pl.BlockSpec((1, N, Cout), lambda b: (b, 0, 0)),
            pl.BlockSpec((1, M, Cout), lambda b: (b, 0, 0)),
            pl.BlockSpec((1, Cout), lambda b: (0, 0)),
            pl.BlockSpec((1, Cout), lambda b: (0, 0)),
            pl.BlockSpec((1, Cout), lambda b: (0, 0)),
            pl.BlockSpec((1, Cout), lambda b: (0, 0)),
        ],
        out_shape=[
            jax.ShapeDtypeStruct((B, N, Cout), jnp.float32),
            jax.ShapeDtypeStruct((B, M, Cout), jnp.float32),
            jax.ShapeDtypeStruct((1, Cout), jnp.float32),
            jax.ShapeDtypeStruct((1, Cout), jnp.float32),
            jax.ShapeDtypeStruct((1, Cout), jnp.float32),
            jax.ShapeDtypeStruct((1, Cout), jnp.float32),
        ],
    )(x1, x2, w_up_t, w_lat_t)


def _nn_body(p2t_ref, p1_ref, iotac_ref, lat_ref, sc2_ref, sh2_ref,
             i0_ref, i1_ref, i2_ref, wt_ref, ylat_ref, *, n_coarse, b0):
    b = pl.program_id(0) + b0
    p2t = p2t_ref[0]
    p1 = p1_ref[0]
    d2 = ((p1[:, 0:1] - p2t[0:1, :]) ** 2
          + (p1[:, 1:2] - p2t[1:2, :]) ** 2
          + (p1[:, 2:3] - p2t[2:3, :]) ** 2)
    iota_c = iotac_ref[...]
    vals, idxs = [], []
    dw = d2
    for k in range(3):
        mv = jnp.min(dw, axis=0, keepdims=True)
        mi = jnp.min(jnp.where(dw == mv, iota_c, jnp.float32(n_coarse)),
                     axis=0, keepdims=True)
        vals.append(mv)
        idxs.append(mi)
        if k < 2:
            dw = jnp.where(iota_c == mi, jnp.float32(jnp.inf), dw)
    recips = [1.0 / (v + 1e-8) for v in vals]
    norm = (recips[0] + recips[1]) + recips[2]
    ws = [r / norm for r in recips]
    wt_ref[0] = jnp.concatenate(ws + [ws[0]] * 5, axis=0)
    i0_ref[0] = idxs[0].astype(jnp.int32) + b * n_coarse
    i1_ref[0] = idxs[1].astype(jnp.int32) + b * n_coarse
    i2_ref[0] = idxs[2].astype(jnp.int32) + b * n_coarse
    ylat_ref[0] = jnp.maximum(lat_ref[0] * sc2_ref[...] + sh2_ref[...], 0.0)


def _nn_search(p2t, p1, iotac, lat_raw, scale2, shift2, mb, b0):
    B, _, M = p2t.shape
    N = p1.shape[1]
    Cout = lat_raw.shape[2]
    body = functools.partial(_nn_body, n_coarse=N, b0=b0)
    return pl.pallas_call(
        body,
        grid=(B, M // mb),
        in_specs=[
            pl.BlockSpec((1, 3, mb), lambda b, j: (b, 0, j)),
            pl.BlockSpec((1, N, 3), lambda b, j: (b, 0, 0)),
            pl.BlockSpec((N, 1), lambda b, j: (0, 0)),
            pl.BlockSpec((1, mb, Cout), lambda b, j: (b, j, 0)),
            pl.BlockSpec((1, Cout), lambda b, j: (0, 0)),
            pl.BlockSpec((1, Cout), lambda b, j: (0, 0)),
        ],
        out_specs=[
            pl.BlockSpec((1, 1, mb), lambda b, j: (b, 0, j)),
            pl.BlockSpec((1, 1, mb), lambda b, j: (b, 0, j)),
            pl.BlockSpec((1, 1, mb), lambda b, j: (b, 0, j)),
            pl.BlockSpec((1, 8, mb), lambda b, j: (b, 0, j)),
            pl.BlockSpec((1, mb, Cout), lambda b, j: (b, j, 0)),
        ],
        out_shape=[
            jax.ShapeDtypeStruct((B, 1, M), jnp.int32),
            jax.ShapeDtypeStruct((B, 1, M), jnp.int32),
            jax.ShapeDtypeStruct((B, 1, M), jnp.int32),
            jax.ShapeDtypeStruct((B, 8, M), jnp.float32),
            jax.ShapeDtypeStruct((B, M, Cout), jnp.float32),
        ],
    )(p2t, p1, iotac, lat_raw, scale2, shift2)


def _hnorm_body(h_ref, sc_ref, sh_ref, out_ref):
    out_ref[0] = jnp.maximum(h_ref[0] * sc_ref[...] + sh_ref[...], 0.0)


def _hnorm(h_raw, scale1, shift1):
    B, N, Cout = h_raw.shape
    return pl.pallas_call(
        _hnorm_body,
        grid=(B,),
        in_specs=[
            pl.BlockSpec((1, N, Cout), lambda b: (b, 0, 0)),
            pl.BlockSpec((1, Cout), lambda b: (0, 0)),
            pl.BlockSpec((1, Cout), lambda b: (0, 0)),
        ],
        out_specs=pl.BlockSpec((1, N, Cout), lambda b: (b, 0, 0)),
        out_shape=jax.ShapeDtypeStruct((B, N, Cout), jnp.float32),
    )(h_raw, scale1, shift1)


def _interp_sc(h_flat, idx0, idx1, idx2, wt_flat, ylat_flat, m_fine):
    BM, Cout = ylat_flat.shape
    M = m_fine
    info = plsc.get_sparse_core_info()
    NC, NS, LANES = info.num_cores, info.num_subcores, info.num_lanes
    NW = NC * NS
    QT = BM // NW
    CH = 32
    NCHUNK = QT // CH
    CVEC = Cout // LANES

    mesh = plsc.VectorSubcoreMesh(core_axis_name="c", subcore_axis_name="s")

    vm = pltpu.VMEM
    dma = pltpu.SemaphoreType.DMA
    scratch = []
    for _ in range(2):
        scratch += [
            vm((CH,), jnp.int32),
            vm((CH,), jnp.int32),
            vm((CH,), jnp.int32),
            vm((CH, Cout), jnp.float32),
            vm((CH, Cout), jnp.float32),
            vm((CH, Cout), jnp.float32),
            vm((CH,), jnp.float32),
            vm((CH,), jnp.float32),
            vm((CH,), jnp.float32),
            vm((CH, Cout), jnp.float32),
            vm((CH, Cout), jnp.float32),
            dma, dma, dma, dma, dma,
        ]

    scratch += [vm((CH, LANES), jnp.int32), dma]

    @functools.partial(
        pl.kernel,
        mesh=mesh,
        out_type=jax.ShapeDtypeStruct((BM, Cout), jnp.float32),
        scratch_types=scratch,
    )
    def _body(h_hbm, i0_hbm, i1_hbm, i2_hbm, wt_hbm, ylat_hbm, qtab_hbm,
              out_hbm, *bufs):
        wid = lax.axis_index("s") * NC + lax.axis_index("c")
        base = wid * QT
        B0, B1 = bufs[:16], bufs[16:32]
        qtab_v, s_q = bufs[32], bufs[33]
        idx_hbms = (i0_hbm, i1_hbm, i2_hbm)

        def wslices(i):
            qb = base + i * CH
            bb = qb // M
            m0 = qb - bb * M
            return [wt_hbm.at[pl.ds(pl.multiple_of((bb * 8 + k) * M + m0, 8),
                                    CH)]
                    for k in range(3)]

        def issue_inputs(i, bset):
            (iv0, iv1, iv2, _, _, _, wv0, wv1, wv2, ylv, _,
             s_i, _, s_w, s_y, _) = bset
            qb = pl.multiple_of(base + i * CH, 8)
            for ih, iv in zip(idx_hbms, (iv0, iv1, iv2)):
                pltpu.async_copy(ih.at[pl.ds(qb, CH)], iv, s_i)
            for ws, wv in zip(wslices(i), (wv0, wv1, wv2)):
                pltpu.async_copy(ws, wv, s_w)
            pltpu.async_copy(ylat_hbm.at[pl.ds(qb, CH)], ylv, s_y)

        def wait_idx(i, bset):
            iv0, iv1, iv2 = bset[0], bset[1], bset[2]
            s_i = bset[11]
            qb = pl.multiple_of(base + i * CH, 8)
            for ih, iv in zip(idx_hbms, (iv0, iv1, iv2)):
                pltpu.make_async_copy(ih.at[pl.ds(qb, CH)], iv, s_i).wait()

        def issue_gather(bset):
            iv0, iv1, iv2, r0, r1, r2 = bset[:6]
            s_g = bset[12]
            pltpu.async_copy(h_hbm.at[iv0], r0, s_g)
            pltpu.async_copy(h_hbm.at[iv1], r1, s_g)
            pltpu.async_copy(h_hbm.at[iv2], r2, s_g)

        def wait_out(i, bset):
            outv, s_o = bset[10], bset[15]
            qb = pl.multiple_of(base + i * CH, 8)
            pltpu.make_async_copy(outv, out_hbm.at[pl.ds(qb, CH)], s_o).wait()

        def run_chunk(i, bset, nxt, prefetch_i):
            (iv0, iv1, iv2, r0, r1, r2, wv0, wv1, wv2, ylv, outv,
             s_i, s_g, s_w, s_y, s_o) = bset
            qb = pl.multiple_of(base + i * CH, 8)

            @pl.when(i + 1 < NCHUNK)
            def _():
                wait_idx(i + 1, nxt)
                issue_gather(nxt)

            for iv, rv in ((iv0, r0), (iv1, r1), (iv2, r2)):
                pltpu.make_async_copy(h_hbm.at[iv], rv, s_g).wait()
            for ws, wv in zip(wslices(i), (wv0, wv1, wv2)):
                pltpu.make_async_copy(ws, wv, s_w).wait()
            pltpu.make_async_copy(ylat_hbm.at[pl.ds(qb, CH)], ylv, s_y).wait()

            @pl.when(i >= 2)
            def _():
                wait_out(i - 2, bset)

            def qloop(q, c2):
                qv = qtab_v[q, :]
                qb16 = (q // LANES) * LANES
                w0 = wv0[pl.ds(qb16, LANES)].at[qv].get(
                    mode="promise_in_bounds")
                w1 = wv1[pl.ds(qb16, LANES)].at[qv].get(
                    mode="promise_in_bounds")
                w2 = wv2[pl.ds(qb16, LANES)].at[qv].get(
                    mode="promise_in_bounds")
                for c in range(CVEC):
                    sl = pl.ds(c * LANES, LANES)
                    acc = (ylv[q, sl]
                           + w0 * r0[q, sl]
                           + w1 * r1[q, sl]
                           + w2 * r2[q, sl])
                    outv[q, sl] = acc
                return c2

            lax.fori_loop(0, CH, qloop, 0)
            pltpu.async_copy(outv, out_hbm.at[pl.ds(qb, CH)], s_o)

            @pl.when(prefetch_i < NCHUNK)
            def _():
                issue_inputs(prefetch_i, bset)

        pltpu.async_copy(qtab_hbm, qtab_v, s_q).wait()
        issue_inputs(0, B0)
        wait_idx(0, B0)
        issue_gather(B0)
        issue_inputs(1, B1)

        def pair(j, carry):
            i0 = j * 2
            run_chunk(i0, B0, B1, i0 + 2)
            run_chunk(i0 + 1, B1, B0, i0 + 3)
            return carry

        lax.fori_loop(0, NCHUNK // 2, pair, 0)

        wait_out(NCHUNK - 2, B0)
        wait_out(NCHUNK - 1, B1)

    qtab = jnp.broadcast_to((jnp.arange(CH, dtype=jnp.int32) % LANES)[:, None],
                            (CH, LANES))
    return _body(h_flat, idx0, idx1, idx2, wt_flat, ylat_flat, qtab)


def kernel(x1, p1, x2, p2, W_up, g_up, b_up, W_lat, g_lat, b_lat):
    B, N, Cin = x1.shape
    M = x2.shape[1]
    Cout = W_up.shape[0]
    eps = 1e-5

    h_raw, lat_raw, s1, ss1, s2, ss2 = _mm_stats(x1, x2, W_up.T, W_lat.T)

    cnt1 = jnp.float32(B * N)
    mean1 = s1 / cnt1
    var1 = ss1 / cnt1 - mean1 * mean1
    scale1 = g_up[None, :] * lax.rsqrt(var1 + eps)
    shift1 = b_up[None, :] - mean1 * scale1

    cnt2 = jnp.float32(B * M)
    mean2 = s2 / cnt2
    var2 = ss2 / cnt2 - mean2 * mean2
    scale2 = g_lat[None, :] * lax.rsqrt(var2 + eps)
    shift2 = b_lat[None, :] - mean2 * scale2

    iotac = jnp.arange(N, dtype=jnp.float32)[:, None]
    p2t = jnp.transpose(p2, (0, 2, 1))
    h_final = _hnorm(h_raw, scale1, shift1)
    h_flat = h_final.reshape(B * N, Cout)

    G = 2
    Bh = B // G
    outs = []
    for g in range(G):
        sl = slice(g * Bh, (g + 1) * Bh)
        i0, i1, i2, wt, ylat = _nn_search(p2t[sl], p1[sl], iotac,
                                          lat_raw[sl], scale2, shift2,
                                          mb=1024, b0=g * Bh)
        outs.append(_interp_sc(h_flat, i0.reshape(Bh * M),
                               i1.reshape(Bh * M), i2.reshape(Bh * M),
                               wt.reshape(Bh * 8 * M),
                               ylat.reshape(Bh * M, Cout), M))
    out = jnp.concatenate(outs, axis=0)
    return out.reshape(B, M, Cout), p2


# speedup vs baseline: 1.0637x; 1.0637x over previous
import functools

import jax
import jax.numpy as jnp
from jax import lax
from jax.experimental import pallas as pl
from jax.experimental.pallas import tpu as pltpu
from jax.experimental.pallas import tpu_sc as plsc


def _mm_stats_body(x1_ref, x2_ref, wup_ref, wlat_ref,
                   h_ref, lat_ref, s1_ref, ss1_ref, s2_ref, ss2_ref):
    b = pl.program_id(0)
    h = jnp.dot(x1_ref[0], wup_ref[...], preferred_element_type=jnp.float32)
    lat = jnp.dot(x2_ref[0], wlat_ref[...], preferred_element_type=jnp.float32)
    h_ref[0] = h
    lat_ref[0] = lat
    s1 = jnp.sum(h, axis=0, keepdims=True)
    ss1 = jnp.sum(h * h, axis=0, keepdims=True)
    s2 = jnp.sum(lat, axis=0, keepdims=True)
    ss2 = jnp.sum(lat * lat, axis=0, keepdims=True)

    @pl.when(b == 0)
    def _():
        s1_ref[...] = s1
        ss1_ref[...] = ss1
        s2_ref[...] = s2
        ss2_ref[...] = ss2

    @pl.when(b != 0)
    def _():
        s1_ref[...] += s1
        ss1_ref[...] += ss1
        s2_ref[...] += s2
        ss2_ref[...] += ss2


def _mm_stats(x1, x2, w_up_t, w_lat_t):
    B, N, Cin = x1.shape
    M = x2.shape[1]
    Cout = w_up_t.shape[1]
    return pl.pallas_call(
        _mm_stats_body,
        grid=(B,),
        in_specs=[
            pl.BlockSpec((1, N, Cin), lambda b: (b, 0, 0)),
            pl.BlockSpec((1, M, Cout), lambda b: (b, 0, 0)),
            pl.BlockSpec((Cin, Cout), lambda b: (0, 0)),
            pl.BlockSpec((Cout, Cout), lambda b: (0, 0)),
        ],
        out_specs=[
            pl.BlockSpec((1, N, Cout), lambda b: (b, 0, 0)),
            pl.BlockSpec((1, M, Cout), lambda b: (b, 0, 0)),
            pl.BlockSpec((1, Cout), lambda b: (0, 0)),
            pl.BlockSpec((1, Cout), lambda b: (0, 0)),
            pl.BlockSpec((1, Cout), lambda b: (0, 0)),
            pl.BlockSpec((1, Cout), lambda b: (0, 0)),
        ],
        out_shape=[
            jax.ShapeDtypeStruct((B, N, Cout), jnp.float32),
            jax.ShapeDtypeStruct((B, M, Cout), jnp.float32),
            jax.ShapeDtypeStruct((1, Cout), jnp.float32),
            jax.ShapeDtypeStruct((1, Cout), jnp.float32),
            jax.ShapeDtypeStruct((1, Cout), jnp.float32),
            jax.ShapeDtypeStruct((1, Cout), jnp.float32),
        ],
    )(x1, x2, w_up_t, w_lat_t)


def _nn_body(p2t_ref, p1_ref, iotac_ref, lat_ref, sc2_ref, sh2_ref,
             i0_ref, i1_ref, i2_ref, wt_ref, ylat_ref, *, n_coarse, b0):
    b = pl.program_id(0) + b0
    p2t = p2t_ref[0]
    p1 = p1_ref[0]
    d2 = ((p1[:, 0:1] - p2t[0:1, :]) ** 2
          + (p1[:, 1:2] - p2t[1:2, :]) ** 2
          + (p1[:, 2:3] - p2t[2:3, :]) ** 2)
    iota_c = iotac_ref[...]
    vals, idxs = [], []
    dw = d2
    for k in range(3):
        mv = jnp.min(dw, axis=0, keepdims=True)
        mi = jnp.min(jnp.where(dw == mv, iota_c, jnp.float32(n_coarse)),
                     axis=0, keepdims=True)
        vals.append(mv)
        idxs.append(mi)
        if k < 2:
            dw = jnp.where(iota_c == mi, jnp.float32(jnp.inf), dw)
    recips = [1.0 / (v + 1e-8) for v in vals]
    norm = (recips[0] + recips[1]) + recips[2]
    ws = [r / norm for r in recips]
    wt_ref[0] = jnp.concatenate(ws + [ws[0]] * 5, axis=0)
    i0_ref[0] = idxs[0].astype(jnp.int32) + b * n_coarse
    i1_ref[0] = idxs[1].astype(jnp.int32) + b * n_coarse
    i2_ref[0] = idxs[2].astype(jnp.int32) + b * n_coarse
    ylat_ref[0] = jnp.maximum(lat_ref[0] * sc2_ref[...] + sh2_ref[...], 0.0)


def _nn_search(p2t, p1, iotac, lat_raw, scale2, shift2, mb, b0):
    B, _, M = p2t.shape
    N = p1.shape[1]
    Cout = lat_raw.shape[2]
    body = functools.partial(_nn_body, n_coarse=N, b0=b0)
    return pl.pallas_call(
        body,
        grid=(B, M // mb),
        in_specs=[
            pl.BlockSpec((1, 3, mb), lambda b, j: (b, 0, j)),
            pl.BlockSpec((1, N, 3), lambda b, j: (b, 0, 0)),
            pl.BlockSpec((N, 1), lambda b, j: (0, 0)),
            pl.BlockSpec((1, mb, Cout), lambda b, j: (b, j, 0)),
            pl.BlockSpec((1, Cout), lambda b, j: (0, 0)),
            pl.BlockSpec((1, Cout), lambda b, j: (0, 0)),
        ],
        out_specs=[
            pl.BlockSpec((1, 1, mb), lambda b, j: (b, 0, j)),
            pl.BlockSpec((1, 1, mb), lambda b, j: (b, 0, j)),
            pl.BlockSpec((1, 1, mb), lambda b, j: (b, 0, j)),
            pl.BlockSpec((1, 8, mb), lambda b, j: (b, 0, j)),
            pl.BlockSpec((1, mb, Cout), lambda b, j: (b, j, 0)),
        ],
        out_shape=[
            jax.ShapeDtypeStruct((B, 1, M), jnp.int32),
            jax.ShapeDtypeStruct((B, 1, M), jnp.int32),
            jax.ShapeDtypeStruct((B, 1, M), jnp.int32),
            jax.ShapeDtypeStruct((B, 8, M), jnp.float32),
            jax.ShapeDtypeStruct((B, M, Cout), jnp.float32),
        ],
    )(p2t, p1, iotac, lat_raw, scale2, shift2)


def _hnorm_body(h_ref, sc_ref, sh_ref, out_ref):
    out_ref[0] = jnp.maximum(h_ref[0] * sc_ref[...] + sh_ref[...], 0.0)


def _hnorm(h_raw, scale1, shift1):
    B, N, Cout = h_raw.shape
    return pl.pallas_call(
        _hnorm_body,
        grid=(B,),
        in_specs=[
            pl.BlockSpec((1, N, Cout), lambda b: (b, 0, 0)),
            pl.BlockSpec((1, Cout), lambda b: (0, 0)),
            pl.BlockSpec((1, Cout), lambda b: (0, 0)),
        ],
        out_specs=pl.BlockSpec((1, N, Cout), lambda b: (b, 0, 0)),
        out_shape=jax.ShapeDtypeStruct((B, N, Cout), jnp.float32),
    )(h_raw, scale1, shift1)


def _interp_sc(h_flat, idx0, idx1, idx2, wt_flat, ylat_flat, m_fine):
    BM, Cout = ylat_flat.shape
    M = m_fine
    info = plsc.get_sparse_core_info()
    NC, NS, LANES = info.num_cores, info.num_subcores, info.num_lanes
    NW = NC * NS
    QT = BM // NW
    CH = 32
    NCHUNK = QT // CH
    CVEC = Cout // LANES

    mesh = plsc.VectorSubcoreMesh(core_axis_name="c", subcore_axis_name="s")

    vm = pltpu.VMEM
    dma = pltpu.SemaphoreType.DMA
    scratch = []
    for _ in range(2):
        scratch += [
            vm((CH,), jnp.int32),
            vm((CH,), jnp.int32),
            vm((CH,), jnp.int32),
            vm((CH, Cout), jnp.float32),
            vm((CH, Cout), jnp.float32),
            vm((CH, Cout), jnp.float32),
            vm((CH,), jnp.float32),
            vm((CH,), jnp.float32),
            vm((CH,), jnp.float32),
            vm((CH, Cout), jnp.float32),
            vm((CH, Cout), jnp.float32),
            dma, dma, dma, dma, dma,
        ]

    scratch += [vm((CH, LANES), jnp.int32), dma]

    @functools.partial(
        pl.kernel,
        mesh=mesh,
        out_type=jax.ShapeDtypeStruct((BM, Cout), jnp.float32),
        scratch_types=scratch,
    )
    def _body(h_hbm, i0_hbm, i1_hbm, i2_hbm, wt_hbm, ylat_hbm, qtab_hbm,
              out_hbm, *bufs):
        wid = lax.axis_index("s") * NC + lax.axis_index("c")
        base = wid * QT
        B0, B1 = bufs[:16], bufs[16:32]
        qtab_v, s_q = bufs[32], bufs[33]
        idx_hbms = (i0_hbm, i1_hbm, i2_hbm)

        def wslices(i):
            qb = base + i * CH
            bb = qb // M
            m0 = qb - bb * M
            return [wt_hbm.at[pl.ds(pl.multiple_of((bb * 8 + k) * M + m0, 8),
                                    CH)]
                    for k in range(3)]

        def issue_inputs(i, bset):
            (iv0, iv1, iv2, _, _, _, wv0, wv1, wv2, ylv, _,
             s_i, _, s_w, s_y, _) = bset
            qb = pl.multiple_of(base + i * CH, 8)
            for ih, iv in zip(idx_hbms, (iv0, iv1, iv2)):
                pltpu.async_copy(ih.at[pl.ds(qb, CH)], iv, s_i)
            for ws, wv in zip(wslices(i), (wv0, wv1, wv2)):
                pltpu.async_copy(ws, wv, s_w)
            pltpu.async_copy(ylat_hbm.at[pl.ds(qb, CH)], ylv, s_y)

        def wait_idx(i, bset):
            iv0, iv1, iv2 = bset[0], bset[1], bset[2]
            s_i = bset[11]
            qb = pl.multiple_of(base + i * CH, 8)
            for ih, iv in zip(idx_hbms, (iv0, iv1, iv2)):
                pltpu.make_async_copy(ih.at[pl.ds(qb, CH)], iv, s_i).wait()

        def issue_gather(bset):
            iv0, iv1, iv2, r0, r1, r2 = bset[:6]
            s_g = bset[12]
            pltpu.async_copy(h_hbm.at[iv0], r0, s_g)
            pltpu.async_copy(h_hbm.at[iv1], r1, s_g)
            pltpu.async_copy(h_hbm.at[iv2], r2, s_g)

        def wait_out(i, bset):
            outv, s_o = bset[10], bset[15]
            qb = pl.multiple_of(base + i * CH, 8)
            pltpu.make_async_copy(outv, out_hbm.at[pl.ds(qb, CH)], s_o).wait()

        def run_chunk(i, bset, nxt, prefetch_i):
            (iv0, iv1, iv2, r0, r1, r2, wv0, wv1, wv2, ylv, outv,
             s_i, s_g, s_w, s_y, s_o) = bset
            qb = pl.multiple_of(base + i * CH, 8)

            @pl.when(i + 1 < NCHUNK)
            def _():
                wait_idx(i + 1, nxt)
                issue_gather(nxt)

            for iv, rv in ((iv0, r0), (iv1, r1), (iv2, r2)):
                pltpu.make_async_copy(h_hbm.at[iv], rv, s_g).wait()
            for ws, wv in zip(wslices(i), (wv0, wv1, wv2)):
                pltpu.make_async_copy(ws, wv, s_w).wait()
            pltpu.make_async_copy(ylat_hbm.at[pl.ds(qb, CH)], ylv, s_y).wait()

            @pl.when(i >= 2)
            def _():
                wait_out(i - 2, bset)

            def qloop(q, c2):
                qv = qtab_v[q, :]
                qb16 = (q // LANES) * LANES
                w0 = wv0[pl.ds(qb16, LANES)].at[qv].get(
                    mode="promise_in_bounds")
                w1 = wv1[pl.ds(qb16, LANES)].at[qv].get(
                    mode="promise_in_bounds")
                w2 = wv2[pl.ds(qb16, LANES)].at[qv].get(
                    mode="promise_in_bounds")
                for c in range(CVEC):
                    sl = pl.ds(c * LANES, LANES)
                    acc = (ylv[q, sl]
                           + w0 * r0[q, sl]
                           + w1 * r1[q, sl]
                           + w2 * r2[q, sl])
                    outv[q, sl] = acc
                return c2

            lax.fori_loop(0, CH, qloop, 0)
            pltpu.async_copy(outv, out_hbm.at[pl.ds(qb, CH)], s_o)

            @pl.when(prefetch_i < NCHUNK)
            def _():
                issue_inputs(prefetch_i, bset)

        pltpu.async_copy(qtab_hbm, qtab_v, s_q).wait()
        issue_inputs(0, B0)
        wait_idx(0, B0)
        issue_gather(B0)
        issue_inputs(1, B1)

        def pair(j, carry):
            i0 = j * 2
            run_chunk(i0, B0, B1, i0 + 2)
            run_chunk(i0 + 1, B1, B0, i0 + 3)
            return carry

        lax.fori_loop(0, NCHUNK // 2, pair, 0)

        wait_out(NCHUNK - 2, B0)
        wait_out(NCHUNK - 1, B1)

    qtab = jnp.broadcast_to((jnp.arange(CH, dtype=jnp.int32) % LANES)[:, None],
                            (CH, LANES))
    return _body(h_flat, idx0, idx1, idx2, wt_flat, ylat_flat, qtab)


def kernel(x1, p1, x2, p2, W_up, g_up, b_up, W_lat, g_lat, b_lat):
    B, N, Cin = x1.shape
    M = x2.shape[1]
    Cout = W_up.shape[0]
    eps = 1e-5

    h_raw, lat_raw, s1, ss1, s2, ss2 = _mm_stats(x1, x2, W_up.T, W_lat.T)

    cnt1 = jnp.float32(B * N)
    mean1 = s1 / cnt1
    var1 = ss1 / cnt1 - mean1 * mean1
    scale1 = g_up[None, :] * lax.rsqrt(var1 + eps)
    shift1 = b_up[None, :] - mean1 * scale1

    cnt2 = jnp.float32(B * M)
    mean2 = s2 / cnt2
    var2 = ss2 / cnt2 - mean2 * mean2
    scale2 = g_lat[None, :] * lax.rsqrt(var2 + eps)
    shift2 = b_lat[None, :] - mean2 * scale2

    iotac = jnp.arange(N, dtype=jnp.float32)[:, None]
    p2t = jnp.transpose(p2, (0, 2, 1))
    h_final = _hnorm(h_raw, scale1, shift1)
    h_flat = h_final.reshape(B * N, Cout)

    G = 1
    Bh = B // G
    outs = []
    for g in range(G):
        sl = slice(g * Bh, (g + 1) * Bh)
        i0, i1, i2, wt, ylat = _nn_search(p2t[sl], p1[sl], iotac,
                                          lat_raw[sl], scale2, shift2,
                                          mb=1024, b0=g * Bh)
        outs.append(_interp_sc(h_flat, i0.reshape(Bh * M),
                               i1.reshape(Bh * M), i2.reshape(Bh * M),
                               wt.reshape(Bh * 8 * M),
                               ylat.reshape(Bh * M, Cout), M))
    out = jnp.concatenate(outs, axis=0)
    return out.reshape(B, M, Cout), p2
